# baseline (speedup 1.0000x reference)
import jax
import jax.numpy as jnp
from jax.experimental import pallas as pl

_BLK = 512


def _mm_mask_kernel(emb_ref, mask_ref, w_ref, out_ref):
    e = emb_ref[...]
    w = w_ref[...]
    logits = jax.lax.dot_general(
        e, w,
        dimension_numbers=(((1,), (1,)), ((), ())),
        preferred_element_type=jnp.float32,
    )
    m = mask_ref[...]
    out_ref[...] = jnp.where(m > 0, logits, -jnp.inf)


def kernel(emb_sentences, att_sentences, W):
    B, L, D = emb_sentences.shape
    NL = W.shape[0]
    R = B * L
    emb = emb_sentences.reshape(R, D)
    mask = att_sentences.reshape(R, 1).astype(jnp.float32)

    out = pl.pallas_call(
        _mm_mask_kernel,
        grid=(R // _BLK,),
        in_specs=[
            pl.BlockSpec((_BLK, D), lambda i: (i, 0)),
            pl.BlockSpec((_BLK, 1), lambda i: (i, 0)),
            pl.BlockSpec((NL, D), lambda i: (0, 0)),
        ],
        out_specs=pl.BlockSpec((_BLK, NL), lambda i: (i, 0)),
        out_shape=jax.ShapeDtypeStruct((R, NL), jnp.float32),
    )(emb, mask, W)
    return out.reshape(B, L, NL)


# speedup vs baseline: 1.2950x; 1.2950x over previous
import jax
import jax.numpy as jnp
from jax.experimental import pallas as pl

_BLK = 2048


def _mm_mask_kernel(emb_ref, mask_ref, w_ref, out_ref):
    e = emb_ref[...]
    w = w_ref[...]
    logits = jax.lax.dot_general(
        e, w,
        dimension_numbers=(((1,), (1,)), ((), ())),
        preferred_element_type=jnp.float32,
    )
    m = mask_ref[...]
    out_ref[...] = jnp.where(m > 0, logits, -jnp.inf)


def kernel(emb_sentences, att_sentences, W):
    B, L, D = emb_sentences.shape
    NL = W.shape[0]
    R = B * L
    emb = emb_sentences.reshape(R, D)
    mask = att_sentences.reshape(R, 1).astype(jnp.float32)

    out = pl.pallas_call(
        _mm_mask_kernel,
        grid=(R // _BLK,),
        in_specs=[
            pl.BlockSpec((_BLK, D), lambda i: (i, 0)),
            pl.BlockSpec((_BLK, 1), lambda i: (i, 0)),
            pl.BlockSpec((NL, D), lambda i: (0, 0)),
        ],
        out_specs=pl.BlockSpec((_BLK, NL), lambda i: (i, 0)),
        out_shape=jax.ShapeDtypeStruct((R, NL), jnp.float32),
    )(emb, mask, W)
    return out.reshape(B, L, NL)
